# baseline (speedup 1.0000x reference)
import functools

import jax
import jax.numpy as jnp
import numpy as np
from jax import lax
from jax.experimental import pallas as pl
from jax.experimental.pallas import tpu as pltpu
from jax.experimental.pallas import tpu_sc as plsc

N_PAD = 10240
ROWS = N_PAD // 16
CHUNK = 128
CH = 80
PH = 2
CHP = CH // PH
E_PAD = 16 * CH * CHUNK
DUMMY = 10200
HALF = 128
NBUF = 2
CHG = 64

_MESH = plsc.VectorSubcoreMesh(core_axis_name="c", subcore_axis_name="s")


def _deg_body(dst_hbm, ones_hbm, zeros_hbm, degc_hbm,
              acc_sh, dst_v, ones_v, dsem):
    c = lax.axis_index("c")
    s = lax.axis_index("s")
    r0 = s * ROWS
    pltpu.sync_copy(zeros_hbm.at[pl.ds(r0, ROWS)], acc_sh.at[pl.ds(r0, ROWS)])
    pltpu.sync_copy(dst_hbm.at[PH * s + c], dst_v)
    pltpu.sync_copy(ones_hbm, ones_v)
    plsc.subcore_barrier()

    @pl.loop(0, CHP)
    def _(j):
        pltpu.async_copy(ones_v, acc_sh.at[dst_v.at[j]], dsem, add=True)

    @pl.loop(0, CHP)
    def _(j):
        pltpu.make_async_copy(ones_v, acc_sh.at[dst_v.at[j]], dsem).wait()

    plsc.subcore_barrier()
    pltpu.sync_copy(acc_sh.at[pl.ds(r0, ROWS)],
                    degc_hbm.at[pl.ds(c * N_PAD + r0, ROWS)])


@jax.jit
def _sc_degrees(dst_p, ones16, zeros16):
    k = pl.kernel(
        _deg_body,
        out_type=jax.ShapeDtypeStruct((2 * N_PAD, 16), jnp.float32),
        mesh=_MESH,
        scratch_types=[
            pltpu.VMEM_SHARED((N_PAD, 16), jnp.float32),
            pltpu.VMEM((CHP, CHUNK), jnp.int32),
            pltpu.VMEM((CHUNK, 16), jnp.float32),
            pltpu.SemaphoreType.DMA,
        ],
    )
    return k(dst_p, ones16, zeros16)


def _mp_body(ga_hbm, gb_hbm, src_hbm, dst_hbm, outa_hbm, outb_hbm,
             acc_sh, src_v, dst_v, buf, gsem):
    c = lax.axis_index("c")
    s = lax.axis_index("s")
    r0 = s * ROWS

    def init(g_hbm):
        pltpu.sync_copy(g_hbm.at[pl.ds(r0, ROWS)], acc_sh.at[pl.ds(r0, ROWS)])

    @pl.when(c == 0)
    def _():
        init(ga_hbm)

    @pl.when(c == 1)
    def _():
        init(gb_hbm)

    plsc.subcore_barrier()

    def edge_loop(g_hbm):
        def wait_g(gj, b, h):
            pltpu.make_async_copy(
                g_hbm.at[src_v.at[gj]],
                buf.at[b, pl.ds(h * CHG, CHG)], gsem.at[2 * b + h]).wait()

        def sync_s(j, b):
            pltpu.sync_copy(buf.at[b], acc_sh.at[dst_v.at[j]], add=True)

        def fire_g(gj, b, h):
            pltpu.async_copy(g_hbm.at[src_v.at[gj]],
                             buf.at[b, pl.ds(h * CHG, CHG)], gsem.at[2 * b + h])

        for p in range(PH):
            pltpu.sync_copy(src_hbm.at[PH * s + p], src_v)
            pltpu.sync_copy(dst_hbm.at[PH * s + p], dst_v)
            for b in range(NBUF):
                for h in range(2):
                    fire_g(2 * b + h, b, h)

            @pl.loop(0, CHP - 2, step=2)
            def _(jb):
                for b in range(NBUF):
                    j = jb + b
                    wait_g(2 * j, b, 0)
                    wait_g(2 * j + 1, b, 1)
                    sync_s(j, b)
                    fire_g(2 * j + 4, b, 0)
                    fire_g(2 * j + 5, b, 1)

            for b in range(NBUF):
                j = CHP - 2 + b
                wait_g(2 * j, b, 0)
                wait_g(2 * j + 1, b, 1)
                sync_s(j, b)

    @pl.when(c == 0)
    def _():
        edge_loop(ga_hbm)

    @pl.when(c == 1)
    def _():
        edge_loop(gb_hbm)

    plsc.subcore_barrier()

    @pl.when(c == 0)
    def _():
        pltpu.sync_copy(acc_sh.at[pl.ds(r0, ROWS)], outa_hbm.at[pl.ds(r0, ROWS)])

    @pl.when(c == 1)
    def _():
        pltpu.sync_copy(acc_sh.at[pl.ds(r0, ROWS)], outb_hbm.at[pl.ds(r0, ROWS)])


@jax.jit
def _sc_message_pass(ga, gb, src_p, dst_p):
    k = pl.kernel(
        _mp_body,
        out_type=(jax.ShapeDtypeStruct((N_PAD, HALF), jnp.float32),
                  jax.ShapeDtypeStruct((N_PAD, HALF), jnp.float32)),
        mesh=_MESH,
        scratch_types=[
            pltpu.VMEM_SHARED((N_PAD, HALF), jnp.float32),
            pltpu.VMEM((2 * CHP, CHG), jnp.int32),
            pltpu.VMEM((CHP, CHUNK), jnp.int32),
            pltpu.VMEM((NBUF, CHUNK, HALF), jnp.float32),
            pltpu.SemaphoreType.DMA((2 * NBUF,)),
        ],
    )
    return k(ga, gb, src_p, dst_p)


_BR = 512
_HIGH = jax.lax.Precision.HIGHEST


def _dinv_of(d0_ref, d1_ref):
    deg = d0_ref[:, :1] + d1_ref[:, :1] + 1.0
    return jax.lax.rsqrt(deg)


def _tc1_body(x_ref, w1_ref, d0_ref, d1_ref, ga_ref, gb_ref):
    dinv = _dinv_of(d0_ref, d1_ref)
    h = jnp.dot(x_ref[...], w1_ref[...], precision=_HIGH)
    g = h * dinv
    ga_ref[...] = g[:, :HALF]
    gb_ref[...] = g[:, HALF:]


def _tc2_body(aa_ref, ab_ref, d0_ref, d1_ref, b1_ref, w2_ref, ga_ref, gb_ref):
    dinv = _dinv_of(d0_ref, d1_ref)
    acc = jnp.concatenate([aa_ref[...], ab_ref[...]], axis=1)
    z = jnp.maximum(acc * dinv + b1_ref[...], 0.0)
    h = jnp.dot(z, w2_ref[...], precision=_HIGH)
    g = h * dinv
    ga_ref[...] = g[:, :HALF]
    gb_ref[...] = g[:, HALF:]


def _tc3_body(aa_ref, ab_ref, d0_ref, d1_ref, b2_ref, out_ref):
    dinv = _dinv_of(d0_ref, d1_ref)
    acc = jnp.concatenate([aa_ref[...], ab_ref[...]], axis=1)
    out_ref[...] = acc * dinv + b2_ref[...]


def _row_spec(w):
    return pl.BlockSpec((_BR, w), lambda i: (i, 0))


def _full_spec(shape):
    return pl.BlockSpec(shape, lambda i: (0,) * len(shape))


@jax.jit
def _tc1(x_pad, w1, d0, d1):
    return pl.pallas_call(
        _tc1_body,
        grid=(N_PAD // _BR,),
        in_specs=[_row_spec(256), _full_spec((256, 256)),
                  _row_spec(16), _row_spec(16)],
        out_specs=(_row_spec(HALF), _row_spec(HALF)),
        out_shape=(jax.ShapeDtypeStruct((N_PAD, HALF), jnp.float32),
                   jax.ShapeDtypeStruct((N_PAD, HALF), jnp.float32)),
    )(x_pad, w1, d0, d1)


@jax.jit
def _tc2(aa, ab, d0, d1, b1, w2):
    return pl.pallas_call(
        _tc2_body,
        grid=(N_PAD // _BR,),
        in_specs=[_row_spec(HALF), _row_spec(HALF),
                  _row_spec(16), _row_spec(16),
                  _full_spec((1, 256)), _full_spec((256, 256))],
        out_specs=(_row_spec(HALF), _row_spec(HALF)),
        out_shape=(jax.ShapeDtypeStruct((N_PAD, HALF), jnp.float32),
                   jax.ShapeDtypeStruct((N_PAD, HALF), jnp.float32)),
    )(aa, ab, d0, d1, b1, w2)


@jax.jit
def _tc3(aa, ab, d0, d1, b2):
    return pl.pallas_call(
        _tc3_body,
        grid=(N_PAD // _BR,),
        in_specs=[_row_spec(HALF), _row_spec(HALF),
                  _row_spec(16), _row_spec(16), _full_spec((1, 256))],
        out_specs=_row_spec(256),
        out_shape=jax.ShapeDtypeStruct((N_PAD, 256), jnp.float32),
    )(aa, ab, d0, d1, b2)


def kernel(x, edge_index, W1, b1, W2, b2):
    n, d = x.shape
    e = edge_index.shape[1]

    src = edge_index[0].astype(jnp.int32)
    dst = edge_index[1].astype(jnp.int32)
    perm = jnp.argsort(src)
    src = src[perm]
    dst = dst[perm]
    pad = E_PAD - e
    src_p = jnp.concatenate(
        [src, jnp.zeros((pad,), jnp.int32)]).reshape(16 * PH, 2 * CHP, CHG)
    dst_p = jnp.concatenate(
        [dst, jnp.full((pad,), DUMMY, jnp.int32)]).reshape(16 * PH, CHP, CHUNK)

    x_pad = jnp.pad(x, ((0, N_PAD - n), (0, 0)))
    ones16 = jnp.asarray(np.eye(CHUNK, 16, dtype=np.float32))
    zeros16 = jnp.zeros((N_PAD, 16), jnp.float32)

    degc = _sc_degrees(dst_p, ones16, zeros16)
    d0, d1 = degc[:N_PAD], degc[N_PAD:]

    ga1, gb1 = _tc1(x_pad, W1, d0, d1)
    aa1, ab1 = _sc_message_pass(ga1, gb1, src_p, dst_p)
    ga2, gb2 = _tc2(aa1, ab1, d0, d1, b1.reshape(1, 256), W2)
    aa2, ab2 = _sc_message_pass(ga2, gb2, src_p, dst_p)
    out = _tc3(aa2, ab2, d0, d1, b2.reshape(1, 256))
    return out[:n]


# speedup vs baseline: 1.2320x; 1.2320x over previous
import functools

import jax
import jax.numpy as jnp
import numpy as np
from jax import lax
from jax.experimental import pallas as pl
from jax.experimental.pallas import tpu as pltpu
from jax.experimental.pallas import tpu_sc as plsc

N_PAD = 10240
ROWS = N_PAD // 16
CHUNK = 128
CH = 80
PH = 2
CHP = CH // PH
E_PAD = 16 * CH * CHUNK
DUMMY = 10200
HALF = 128
NBUF = 2
CHG = 64

_MESH = plsc.VectorSubcoreMesh(core_axis_name="c", subcore_axis_name="s")


def _deg_body(dst_hbm, ones_hbm, zeros_hbm, degc_hbm,
              acc_sh, dst_v, ones_v, dsem):
    c = lax.axis_index("c")
    s = lax.axis_index("s")
    r0 = s * ROWS
    pltpu.sync_copy(zeros_hbm.at[pl.ds(r0, ROWS)], acc_sh.at[pl.ds(r0, ROWS)])
    pltpu.sync_copy(dst_hbm.at[PH * s + c], dst_v)
    pltpu.sync_copy(ones_hbm, ones_v)
    plsc.subcore_barrier()

    @pl.loop(0, CHP)
    def _(j):
        pltpu.async_copy(ones_v, acc_sh.at[dst_v.at[j]], dsem, add=True)

    @pl.loop(0, CHP)
    def _(j):
        pltpu.make_async_copy(ones_v, acc_sh.at[dst_v.at[j]], dsem).wait()

    plsc.subcore_barrier()
    pltpu.sync_copy(acc_sh.at[pl.ds(r0, ROWS)],
                    degc_hbm.at[pl.ds(c * N_PAD + r0, ROWS)])


@jax.jit
def _sc_degrees(dst_p, ones16, zeros16):
    k = pl.kernel(
        _deg_body,
        out_type=jax.ShapeDtypeStruct((2 * N_PAD, 16), jnp.float32),
        mesh=_MESH,
        scratch_types=[
            pltpu.VMEM_SHARED((N_PAD, 16), jnp.float32),
            pltpu.VMEM((CHP, CHUNK), jnp.int32),
            pltpu.VMEM((CHUNK, 16), jnp.float32),
            pltpu.SemaphoreType.DMA,
        ],
    )
    return k(dst_p, ones16, zeros16)


def _mp_body(ga_hbm, gb_hbm, src_hbm, dst_hbm, outa_hbm, outb_hbm,
             acc_sh, src_v, dst_v, buf, gsem):
    c = lax.axis_index("c")
    s = lax.axis_index("s")
    r0 = s * ROWS

    def init(g_hbm):
        pltpu.sync_copy(g_hbm.at[pl.ds(r0, ROWS)], acc_sh.at[pl.ds(r0, ROWS)])

    @pl.when(c == 0)
    def _():
        init(ga_hbm)

    @pl.when(c == 1)
    def _():
        init(gb_hbm)

    plsc.subcore_barrier()

    def edge_loop(g_hbm):
        def wait_g(j, b):
            pltpu.make_async_copy(
                g_hbm.at[src_v.at[j]], buf.at[b], gsem.at[b]).wait()

        def sync_s(j, b):
            pltpu.sync_copy(buf.at[b], acc_sh.at[dst_v.at[j]], add=True)

        def fire_g(j, b):
            pltpu.async_copy(g_hbm.at[src_v.at[j]], buf.at[b], gsem.at[b])

        for p in range(PH):
            pltpu.sync_copy(src_hbm.at[PH * s + p], src_v)
            pltpu.sync_copy(dst_hbm.at[PH * s + p], dst_v)
            fire_g(0, 0)
            fire_g(1, 1)

            @pl.loop(0, CHP - 2, step=2)
            def _(jb):
                wait_g(jb, 0)
                sync_s(jb, 0)
                fire_g(jb + 2, 0)
                wait_g(jb + 1, 1)
                sync_s(jb + 1, 1)
                fire_g(jb + 3, 1)

            wait_g(CHP - 2, 0)
            sync_s(CHP - 2, 0)
            wait_g(CHP - 1, 1)
            sync_s(CHP - 1, 1)

    @pl.when(c == 0)
    def _():
        edge_loop(ga_hbm)

    @pl.when(c == 1)
    def _():
        edge_loop(gb_hbm)

    plsc.subcore_barrier()

    @pl.when(c == 0)
    def _():
        pltpu.sync_copy(acc_sh.at[pl.ds(r0, ROWS)], outa_hbm.at[pl.ds(r0, ROWS)])

    @pl.when(c == 1)
    def _():
        pltpu.sync_copy(acc_sh.at[pl.ds(r0, ROWS)], outb_hbm.at[pl.ds(r0, ROWS)])


@jax.jit
def _sc_message_pass(ga, gb, src_p, dst_p):
    k = pl.kernel(
        _mp_body,
        out_type=(jax.ShapeDtypeStruct((N_PAD, HALF), jnp.float32),
                  jax.ShapeDtypeStruct((N_PAD, HALF), jnp.float32)),
        mesh=_MESH,
        scratch_types=[
            pltpu.VMEM_SHARED((N_PAD, HALF), jnp.float32),
            pltpu.VMEM((CHP, CHUNK), jnp.int32),
            pltpu.VMEM((CHP, CHUNK), jnp.int32),
            pltpu.VMEM((NBUF, CHUNK, HALF), jnp.float32),
            pltpu.SemaphoreType.DMA((NBUF,)),
        ],
    )
    return k(ga, gb, src_p, dst_p)


_BR = 512
_HIGH = jax.lax.Precision.HIGHEST


def _dinv_of(d0_ref, d1_ref):
    deg = d0_ref[:, :1] + d1_ref[:, :1] + 1.0
    return jax.lax.rsqrt(deg)


def _tc1_body(x_ref, w1_ref, d0_ref, d1_ref, ga_ref, gb_ref):
    dinv = _dinv_of(d0_ref, d1_ref)
    h = jnp.dot(x_ref[...], w1_ref[...], precision=_HIGH)
    g = h * dinv
    ga_ref[...] = g[:, :HALF]
    gb_ref[...] = g[:, HALF:]


def _tc2_body(aa_ref, ab_ref, d0_ref, d1_ref, b1_ref, w2_ref, ga_ref, gb_ref):
    dinv = _dinv_of(d0_ref, d1_ref)
    acc = jnp.concatenate([aa_ref[...], ab_ref[...]], axis=1)
    z = jnp.maximum(acc * dinv + b1_ref[...], 0.0)
    h = jnp.dot(z, w2_ref[...], precision=_HIGH)
    g = h * dinv
    ga_ref[...] = g[:, :HALF]
    gb_ref[...] = g[:, HALF:]


def _tc3_body(aa_ref, ab_ref, d0_ref, d1_ref, b2_ref, out_ref):
    dinv = _dinv_of(d0_ref, d1_ref)
    acc = jnp.concatenate([aa_ref[...], ab_ref[...]], axis=1)
    out_ref[...] = acc * dinv + b2_ref[...]


def _row_spec(w):
    return pl.BlockSpec((_BR, w), lambda i: (i, 0))


def _full_spec(shape):
    return pl.BlockSpec(shape, lambda i: (0,) * len(shape))


@jax.jit
def _tc1(x_pad, w1, d0, d1):
    return pl.pallas_call(
        _tc1_body,
        grid=(N_PAD // _BR,),
        in_specs=[_row_spec(256), _full_spec((256, 256)),
                  _row_spec(16), _row_spec(16)],
        out_specs=(_row_spec(HALF), _row_spec(HALF)),
        out_shape=(jax.ShapeDtypeStruct((N_PAD, HALF), jnp.float32),
                   jax.ShapeDtypeStruct((N_PAD, HALF), jnp.float32)),
    )(x_pad, w1, d0, d1)


@jax.jit
def _tc2(aa, ab, d0, d1, b1, w2):
    return pl.pallas_call(
        _tc2_body,
        grid=(N_PAD // _BR,),
        in_specs=[_row_spec(HALF), _row_spec(HALF),
                  _row_spec(16), _row_spec(16),
                  _full_spec((1, 256)), _full_spec((256, 256))],
        out_specs=(_row_spec(HALF), _row_spec(HALF)),
        out_shape=(jax.ShapeDtypeStruct((N_PAD, HALF), jnp.float32),
                   jax.ShapeDtypeStruct((N_PAD, HALF), jnp.float32)),
    )(aa, ab, d0, d1, b1, w2)


@jax.jit
def _tc3(aa, ab, d0, d1, b2):
    return pl.pallas_call(
        _tc3_body,
        grid=(N_PAD // _BR,),
        in_specs=[_row_spec(HALF), _row_spec(HALF),
                  _row_spec(16), _row_spec(16), _full_spec((1, 256))],
        out_specs=_row_spec(256),
        out_shape=jax.ShapeDtypeStruct((N_PAD, 256), jnp.float32),
    )(aa, ab, d0, d1, b2)


def kernel(x, edge_index, W1, b1, W2, b2):
    n, d = x.shape
    e = edge_index.shape[1]

    src = edge_index[0].astype(jnp.int32)
    dst = edge_index[1].astype(jnp.int32)
    pad = E_PAD - e
    src_p = jnp.concatenate(
        [src, jnp.zeros((pad,), jnp.int32)]).reshape(16 * PH, CHP, CHUNK)
    dst_p = jnp.concatenate(
        [dst, jnp.full((pad,), DUMMY, jnp.int32)]).reshape(16 * PH, CHP, CHUNK)

    x_pad = jnp.pad(x, ((0, N_PAD - n), (0, 0)))
    ones16 = jnp.asarray(np.eye(CHUNK, 16, dtype=np.float32))
    zeros16 = jnp.zeros((N_PAD, 16), jnp.float32)

    degc = _sc_degrees(dst_p, ones16, zeros16)
    d0, d1 = degc[:N_PAD], degc[N_PAD:]

    ga1, gb1 = _tc1(x_pad, W1, d0, d1)
    aa1, ab1 = _sc_message_pass(ga1, gb1, src_p, dst_p)
    ga2, gb2 = _tc2(aa1, ab1, d0, d1, b1.reshape(1, 256), W2)
    aa2, ab2 = _sc_message_pass(ga2, gb2, src_p, dst_p)
    out = _tc3(aa2, ab2, d0, d1, b2.reshape(1, 256))
    return out[:n]


# speedup vs baseline: 1.2796x; 1.0387x over previous
import functools

import jax
import jax.numpy as jnp
import numpy as np
from jax import lax
from jax.experimental import pallas as pl
from jax.experimental.pallas import tpu as pltpu
from jax.experimental.pallas import tpu_sc as plsc

N_PAD = 10240
ROWS = N_PAD // 16
CHUNK = 128
CH = 80
PH = 2
CHP = CH // PH
E_PAD = 16 * CH * CHUNK
DUMMY = 10200
HALF = 128
NBUF = 2
CHG = 64

_MESH = plsc.VectorSubcoreMesh(core_axis_name="c", subcore_axis_name="s")


def _deg_body(dst_hbm, ones_hbm, zeros_hbm, degc_hbm,
              acc_sh, dst_v, ones_v, dsem):
    c = lax.axis_index("c")
    s = lax.axis_index("s")
    r0 = s * ROWS
    pltpu.sync_copy(zeros_hbm.at[pl.ds(r0, ROWS)], acc_sh.at[pl.ds(r0, ROWS)])
    pltpu.sync_copy(dst_hbm.at[PH * s + c], dst_v)
    pltpu.sync_copy(ones_hbm, ones_v)
    plsc.subcore_barrier()

    @pl.loop(0, CHP)
    def _(j):
        pltpu.async_copy(ones_v, acc_sh.at[dst_v.at[j]], dsem, add=True)

    @pl.loop(0, CHP)
    def _(j):
        pltpu.make_async_copy(ones_v, acc_sh.at[dst_v.at[j]], dsem).wait()

    plsc.subcore_barrier()
    pltpu.sync_copy(acc_sh.at[pl.ds(r0, ROWS)],
                    degc_hbm.at[pl.ds(c * N_PAD + r0, ROWS)])


@jax.jit
def _sc_degrees(dst_p, ones16, zeros16):
    k = pl.kernel(
        _deg_body,
        out_type=jax.ShapeDtypeStruct((2 * N_PAD, 16), jnp.float32),
        mesh=_MESH,
        scratch_types=[
            pltpu.VMEM_SHARED((N_PAD, 16), jnp.float32),
            pltpu.VMEM((CHP, CHUNK), jnp.int32),
            pltpu.VMEM((CHUNK, 16), jnp.float32),
            pltpu.SemaphoreType.DMA,
        ],
    )
    return k(dst_p, ones16, zeros16)


def _mp_body(ga_hbm, gb_hbm, src_hbm, dst_hbm, outa_hbm, outb_hbm,
             acc_sh, src_v, dst_v, buf, gsem):
    c = lax.axis_index("c")
    s = lax.axis_index("s")
    r0 = s * ROWS

    def init(g_hbm):
        pltpu.sync_copy(g_hbm.at[pl.ds(r0, ROWS)], acc_sh.at[pl.ds(r0, ROWS)])

    @pl.when(c == 0)
    def _():
        init(ga_hbm)

    @pl.when(c == 1)
    def _():
        init(gb_hbm)

    plsc.subcore_barrier()

    def edge_loop(g_hbm):
        def wait_g(j, b):
            pltpu.make_async_copy(
                g_hbm.at[src_v.at[j]], buf.at[b], gsem.at[b]).wait()

        def sync_s(j, b):
            pltpu.sync_copy(buf.at[b], acc_sh.at[dst_v.at[j]], add=True)

        def fire_g(j, b):
            pltpu.async_copy(g_hbm.at[src_v.at[j]], buf.at[b], gsem.at[b])

        for p in range(PH):
            pltpu.sync_copy(src_hbm.at[PH * s + p], src_v)
            pltpu.sync_copy(dst_hbm.at[PH * s + p], dst_v)
            fire_g(0, 0)
            fire_g(1, 1)

            @pl.loop(0, CHP - 2, step=2)
            def _(jb):
                wait_g(jb, 0)
                sync_s(jb, 0)
                fire_g(jb + 2, 0)
                wait_g(jb + 1, 1)
                sync_s(jb + 1, 1)
                fire_g(jb + 3, 1)

            wait_g(CHP - 2, 0)
            sync_s(CHP - 2, 0)
            wait_g(CHP - 1, 1)
            sync_s(CHP - 1, 1)

    @pl.when(c == 0)
    def _():
        edge_loop(ga_hbm)

    @pl.when(c == 1)
    def _():
        edge_loop(gb_hbm)

    plsc.subcore_barrier()

    @pl.when(c == 0)
    def _():
        pltpu.sync_copy(acc_sh.at[pl.ds(r0, ROWS)], outa_hbm.at[pl.ds(r0, ROWS)])

    @pl.when(c == 1)
    def _():
        pltpu.sync_copy(acc_sh.at[pl.ds(r0, ROWS)], outb_hbm.at[pl.ds(r0, ROWS)])


@jax.jit
def _sc_message_pass(ga, gb, src_p, dst_p):
    k = pl.kernel(
        _mp_body,
        out_type=(jax.ShapeDtypeStruct((N_PAD, HALF), jnp.float32),
                  jax.ShapeDtypeStruct((N_PAD, HALF), jnp.float32)),
        mesh=_MESH,
        scratch_types=[
            pltpu.VMEM_SHARED((N_PAD, HALF), jnp.float32),
            pltpu.VMEM((CHP, CHUNK), jnp.int32),
            pltpu.VMEM((CHP, CHUNK), jnp.int32),
            pltpu.VMEM((NBUF, CHUNK, HALF), jnp.float32),
            pltpu.SemaphoreType.DMA((NBUF,)),
        ],
    )
    return k(ga, gb, src_p, dst_p)


_BR = 512
_HIGH = jax.lax.Precision.HIGHEST


def _dinv_of(d0_ref, d1_ref):
    deg = d0_ref[:, :1] + d1_ref[:, :1] + 1.0
    return jax.lax.rsqrt(deg)


def _tc1_body(x_ref, w1_ref, d0_ref, d1_ref, ga_ref, gb_ref):
    dinv = _dinv_of(d0_ref, d1_ref)
    h = jnp.dot(x_ref[...], w1_ref[...], precision=_HIGH)
    g = h * dinv
    ga_ref[...] = g[:, :HALF]
    gb_ref[...] = g[:, HALF:]


def _tc2_body(aa_ref, ab_ref, d0_ref, d1_ref, b1_ref, w2_ref, ga_ref, gb_ref):
    dinv = _dinv_of(d0_ref, d1_ref)
    acc = jnp.concatenate([aa_ref[...], ab_ref[...]], axis=1)
    z = jnp.maximum(acc * dinv + b1_ref[...], 0.0)
    h = jnp.dot(z, w2_ref[...], precision=_HIGH)
    g = h * dinv
    ga_ref[...] = g[:, :HALF]
    gb_ref[...] = g[:, HALF:]


def _tc3_body(aa_ref, ab_ref, d0_ref, d1_ref, b2_ref, out_ref):
    dinv = _dinv_of(d0_ref, d1_ref)
    acc = jnp.concatenate([aa_ref[...], ab_ref[...]], axis=1)
    out_ref[...] = acc * dinv + b2_ref[...]


def _row_spec(w):
    return pl.BlockSpec((_BR, w), lambda i: (i, 0))


def _full_spec(shape):
    return pl.BlockSpec(shape, lambda i: (0,) * len(shape))


@jax.jit
def _tc1(x_pad, w1, d0, d1):
    return pl.pallas_call(
        _tc1_body,
        grid=(N_PAD // _BR,),
        in_specs=[_row_spec(256), _full_spec((256, 256)),
                  _row_spec(16), _row_spec(16)],
        out_specs=(_row_spec(HALF), _row_spec(HALF)),
        out_shape=(jax.ShapeDtypeStruct((N_PAD, HALF), jnp.float32),
                   jax.ShapeDtypeStruct((N_PAD, HALF), jnp.float32)),
    )(x_pad, w1, d0, d1)


@jax.jit
def _tc2(aa, ab, d0, d1, b1, w2):
    return pl.pallas_call(
        _tc2_body,
        grid=(N_PAD // _BR,),
        in_specs=[_row_spec(HALF), _row_spec(HALF),
                  _row_spec(16), _row_spec(16),
                  _full_spec((1, 256)), _full_spec((256, 256))],
        out_specs=(_row_spec(HALF), _row_spec(HALF)),
        out_shape=(jax.ShapeDtypeStruct((N_PAD, HALF), jnp.float32),
                   jax.ShapeDtypeStruct((N_PAD, HALF), jnp.float32)),
    )(aa, ab, d0, d1, b1, w2)


@jax.jit
def _tc3(aa, ab, d0, d1, b2):
    br = 400
    spec = lambda w: pl.BlockSpec((br, w), lambda i: (i, 0))
    return pl.pallas_call(
        _tc3_body,
        grid=(10000 // br,),
        in_specs=[spec(HALF), spec(HALF),
                  spec(16), spec(16), _full_spec((1, 256))],
        out_specs=spec(256),
        out_shape=jax.ShapeDtypeStruct((10000, 256), jnp.float32),
    )(aa, ab, d0, d1, b2)


def kernel(x, edge_index, W1, b1, W2, b2):
    n, d = x.shape
    e = edge_index.shape[1]

    src = edge_index[0].astype(jnp.int32)
    dst = edge_index[1].astype(jnp.int32)
    pad = E_PAD - e
    src_p = jnp.concatenate(
        [src, jnp.zeros((pad,), jnp.int32)]).reshape(16 * PH, CHP, CHUNK)
    dst_p = jnp.concatenate(
        [dst, jnp.full((pad,), DUMMY, jnp.int32)]).reshape(16 * PH, CHP, CHUNK)

    x_pad = jnp.pad(x, ((0, N_PAD - n), (0, 0)))
    ones16 = jnp.asarray(np.eye(CHUNK, 16, dtype=np.float32))
    zeros16 = jnp.zeros((N_PAD, 16), jnp.float32)

    degc = _sc_degrees(dst_p, ones16, zeros16)
    d0, d1 = degc[:N_PAD], degc[N_PAD:]

    ga1, gb1 = _tc1(x_pad, W1, d0, d1)
    aa1, ab1 = _sc_message_pass(ga1, gb1, src_p, dst_p)
    ga2, gb2 = _tc2(aa1, ab1, d0, d1, b1.reshape(1, 256), W2)
    aa2, ab2 = _sc_message_pass(ga2, gb2, src_p, dst_p)
    return _tc3(aa2, ab2, d0, d1, b2.reshape(1, 256))
